# Initial kernel scaffold; baseline (speedup 1.0000x reference)
import jax
import jax.numpy as jnp
from jax.experimental import pallas as pl


def kernel(x, gate_w, gate_b, local_w, local_b, W1, b1, W2, b2):
    raise NotImplementedError("write your pallas kernel here")


# speedup vs baseline: 2.9872x; 2.9872x over previous
import functools
import jax
import jax.numpy as jnp
from jax.experimental import pallas as pl

_B, _S, _D = 2, 2048, 1024
_E = 8
_F = 1024
_TOP_K = 2
_ACTIVE_K = 8
_BS = _B * _S
_NSEL = _B * _ACTIVE_K

_TS1 = 512
_TS3 = 512
_TS5 = 512
_FC = 512


def _gating_body(x_ref, gw_ref, gb_ref, lw_ref, lb_ref, gs_ref, ls_ref):
    xt = x_ref[...]
    logits = jnp.dot(xt, gw_ref[...], preferred_element_type=jnp.float32)
    logits = logits + gb_ref[...]
    m = jnp.max(logits, axis=-1, keepdims=True)
    p = jnp.exp(logits - m)
    p = p / jnp.sum(p, axis=-1, keepdims=True)
    eidx = jax.lax.broadcasted_iota(jnp.int32, p.shape, 1)
    m1 = jnp.max(p, axis=-1, keepdims=True)
    i1 = jnp.min(jnp.where(p == m1, eidx, _E), axis=-1, keepdims=True)
    p2 = jnp.where(eidx == i1, -jnp.inf, p)
    m2 = jnp.max(p2, axis=-1, keepdims=True)
    i2 = jnp.min(jnp.where(p2 == m2, eidx, _E), axis=-1, keepdims=True)
    mask = (eidx == i1) | (eidx == i2)
    gs = jnp.where(mask, p, 0.0)
    gs = gs / (jnp.sum(gs, axis=-1, keepdims=True) + 1e-9)
    gs_ref[...] = gs
    ls_ref[...] = (
        jnp.dot(xt, lw_ref[...], preferred_element_type=jnp.float32) + lb_ref[...]
    )


def _topk_body(ls_ref, idx_ref):
    ls = ls_ref[...]
    sidx = jax.lax.broadcasted_iota(jnp.int32, ls.shape, 1)
    cols = []
    for _ in range(_ACTIVE_K):
        m = jnp.max(ls, axis=-1, keepdims=True)
        i = jnp.min(jnp.where(ls == m, sidx, _S), axis=-1, keepdims=True)
        cols.append(i)
        ls = jnp.where(sidx == i, -jnp.inf, ls)
    idx = jnp.concatenate(cols, axis=-1)
    base = jax.lax.broadcasted_iota(jnp.int32, idx.shape, 0) * _S
    idx_ref[...] = idx + base


def _gather_body(x_ref, gs_ref, gidx_ref, xs_ref, gss_ref):
    i = pl.program_id(0)
    row = i * _TS3 + jax.lax.broadcasted_iota(jnp.int32, (_TS3, _NSEL), 0)
    onehot = (row == gidx_ref[...]).astype(jnp.float32)
    dims = (((0,), (0,)), ((), ()))
    px = jax.lax.dot_general(onehot, x_ref[...], dims,
                             preferred_element_type=jnp.float32)
    pg = jax.lax.dot_general(onehot, gs_ref[...], dims,
                             preferred_element_type=jnp.float32)

    @pl.when(i == 0)
    def _():
        xs_ref[...] = jnp.zeros_like(xs_ref)
        gss_ref[...] = jnp.zeros_like(gss_ref)

    xs_ref[...] += px
    gss_ref[...] += pg


def _expert_body(xs_ref, gss_ref, w1_ref, b1_ref, w2_ref, b2_ref, out_ref):
    e = pl.program_id(0)
    fc = pl.program_id(1)
    h = jnp.dot(xs_ref[...], w1_ref[0], preferred_element_type=jnp.float32)
    h = jnp.maximum(h + b1_ref[0], 0.0)
    eo = jnp.dot(h, w2_ref[0], preferred_element_type=jnp.float32)
    gss = gss_ref[...]
    eidx = jax.lax.broadcasted_iota(jnp.int32, gss.shape, 1)
    g = jnp.sum(jnp.where(eidx == e, gss, 0.0), axis=1, keepdims=True)

    @pl.when((e == 0) & (fc == 0))
    def _():
        out_ref[...] = jnp.zeros_like(out_ref)

    acc = g * eo

    @pl.when(fc == 0)
    def _():
        out_ref[...] += g * b2_ref[0]

    out_ref[...] += acc


def _scatter_body(gidx_ref, os_ref, out_ref):
    i = pl.program_id(0)
    row = i * _TS5 + jax.lax.broadcasted_iota(jnp.int32, (_TS5, _NSEL), 0)
    onehot = (row == gidx_ref[...]).astype(jnp.float32)
    out_ref[...] = jnp.dot(onehot, os_ref[...],
                           preferred_element_type=jnp.float32)


@jax.jit
def kernel(x, gate_w, gate_b, local_w, local_b, W1, b1, W2, b2):
    xf = x.reshape(_BS, _D)
    gb2 = gate_b.reshape(1, _E)
    lb2 = local_b.reshape(1, 1)

    gs_flat, ls_flat = pl.pallas_call(
        _gating_body,
        grid=(_BS // _TS1,),
        in_specs=[
            pl.BlockSpec((_TS1, _D), lambda i: (i, 0)),
            pl.BlockSpec((_D, _E), lambda i: (0, 0)),
            pl.BlockSpec((1, _E), lambda i: (0, 0)),
            pl.BlockSpec((_D, 1), lambda i: (0, 0)),
            pl.BlockSpec((1, 1), lambda i: (0, 0)),
        ],
        out_specs=[
            pl.BlockSpec((_TS1, _E), lambda i: (i, 0)),
            pl.BlockSpec((_TS1, 1), lambda i: (i, 0)),
        ],
        out_shape=[
            jax.ShapeDtypeStruct((_BS, _E), jnp.float32),
            jax.ShapeDtypeStruct((_BS, 1), jnp.float32),
        ],
    )(xf, gate_w, gb2, local_w, lb2)

    gidx2 = pl.pallas_call(
        _topk_body,
        in_specs=[pl.BlockSpec((_B, _S), lambda: (0, 0))],
        out_specs=pl.BlockSpec((_B, _ACTIVE_K), lambda: (0, 0)),
        out_shape=jax.ShapeDtypeStruct((_B, _ACTIVE_K), jnp.int32),
    )(ls_flat.reshape(_B, _S))
    gidx = gidx2.reshape(1, _NSEL)

    x_sel, gs_sel = pl.pallas_call(
        _gather_body,
        grid=(_BS // _TS3,),
        in_specs=[
            pl.BlockSpec((_TS3, _D), lambda i: (i, 0)),
            pl.BlockSpec((_TS3, _E), lambda i: (i, 0)),
            pl.BlockSpec((1, _NSEL), lambda i: (0, 0)),
        ],
        out_specs=[
            pl.BlockSpec((_NSEL, _D), lambda i: (0, 0)),
            pl.BlockSpec((_NSEL, _E), lambda i: (0, 0)),
        ],
        out_shape=[
            jax.ShapeDtypeStruct((_NSEL, _D), jnp.float32),
            jax.ShapeDtypeStruct((_NSEL, _E), jnp.float32),
        ],
    )(xf, gs_flat, gidx)

    out_sel = pl.pallas_call(
        _expert_body,
        grid=(_E, _F // _FC),
        in_specs=[
            pl.BlockSpec((_NSEL, _D), lambda e, f: (0, 0)),
            pl.BlockSpec((_NSEL, _E), lambda e, f: (0, 0)),
            pl.BlockSpec((1, _D, _FC), lambda e, f: (e, 0, f)),
            pl.BlockSpec((1, 1, _FC), lambda e, f: (e, 0, f)),
            pl.BlockSpec((1, _FC, _F), lambda e, f: (e, f, 0)),
            pl.BlockSpec((1, 1, _F), lambda e, f: (e, 0, 0)),
        ],
        out_specs=pl.BlockSpec((_NSEL, _F), lambda e, f: (0, 0)),
        out_shape=jax.ShapeDtypeStruct((_NSEL, _F), jnp.float32),
    )(x_sel, gs_sel, W1, b1.reshape(_E, 1, _F), W2, b2.reshape(_E, 1, _F))

    out_flat = pl.pallas_call(
        _scatter_body,
        grid=(_BS // _TS5,),
        in_specs=[
            pl.BlockSpec((1, _NSEL), lambda i: (0, 0)),
            pl.BlockSpec((_NSEL, _F), lambda i: (0, 0)),
        ],
        out_specs=pl.BlockSpec((_TS5, _F), lambda i: (i, 0)),
        out_shape=jax.ShapeDtypeStruct((_BS, _F), jnp.float32),
    )(gidx, out_sel)

    return out_flat.reshape(_B, _S, _F), gs_flat.reshape(_B, _S, _E)


# speedup vs baseline: 3.5625x; 1.1926x over previous
import jax
import jax.numpy as jnp
from jax.experimental import pallas as pl

_B, _S, _D = 2, 2048, 1024
_E = 8
_F = 1024
_ACTIVE_K = 8
_BS = _B * _S
_NSEL = _B * _ACTIVE_K

_TS5 = 512
_FC = 512


def _gating_body(x_ref, gw_ref, gb_ref, lw_ref, lb_ref,
                 gs_ref, gidx_ref, xs_ref, gss_ref):
    xt = x_ref[...]
    logits = jnp.dot(xt, gw_ref[...], preferred_element_type=jnp.float32)
    logits = logits + gb_ref[...]
    m = jnp.max(logits, axis=-1, keepdims=True)
    p = jnp.exp(logits - m)
    p = p / jnp.sum(p, axis=-1, keepdims=True)
    eidx = jax.lax.broadcasted_iota(jnp.int32, p.shape, 1)
    m1 = jnp.max(p, axis=-1, keepdims=True)
    i1 = jnp.min(jnp.where(p == m1, eidx, _E), axis=-1, keepdims=True)
    p2 = jnp.where(eidx == i1, -jnp.inf, p)
    m2 = jnp.max(p2, axis=-1, keepdims=True)
    i2 = jnp.min(jnp.where(p2 == m2, eidx, _E), axis=-1, keepdims=True)
    mask = (eidx == i1) | (eidx == i2)
    gs = jnp.where(mask, p, 0.0)
    gs = gs / (jnp.sum(gs, axis=-1, keepdims=True) + 1e-9)
    gs_ref[...] = gs

    ls = jnp.dot(xt, lw_ref[...], preferred_element_type=jnp.float32)
    ls = ls + lb_ref[...]
    lsT = jnp.transpose(ls)
    col = jax.lax.broadcasted_iota(jnp.int32, (1, _BS), 1)
    pos = jax.lax.broadcasted_iota(jnp.int32, (1, _NSEL), 1)
    gidx = jnp.zeros((1, _NSEL), jnp.int32)
    for b in range(_B):
        work = jnp.where((col >= b * _S) & (col < (b + 1) * _S), lsT, -jnp.inf)
        for k in range(_ACTIVE_K):
            lm = jnp.max(work, axis=-1, keepdims=True)
            gi = jnp.min(jnp.where(work == lm, col, _BS),
                         axis=-1, keepdims=True)
            gidx = jnp.where(pos == (b * _ACTIVE_K + k), gi, gidx)
            work = jnp.where(col == gi, -jnp.inf, work)
    gidx_ref[...] = gidx

    colr = jax.lax.broadcasted_iota(jnp.int32, (_NSEL, _BS), 1)
    onehot = (colr == jnp.transpose(gidx)).astype(jnp.float32)
    xs_ref[...] = jnp.dot(onehot, xt, preferred_element_type=jnp.float32)
    gss_ref[...] = jnp.dot(onehot, gs, preferred_element_type=jnp.float32)


def _expert_body(xs_ref, gss_ref, w1_ref, b1_ref, w2_ref, b2_ref, out_ref):
    e = pl.program_id(0)
    fc = pl.program_id(1)
    h = jnp.dot(xs_ref[...], w1_ref[0], preferred_element_type=jnp.float32)
    h = jnp.maximum(h + b1_ref[0], 0.0)
    eo = jnp.dot(h, w2_ref[0], preferred_element_type=jnp.float32)
    gss = gss_ref[...]
    eidx = jax.lax.broadcasted_iota(jnp.int32, gss.shape, 1)
    g = jnp.sum(jnp.where(eidx == e, gss, 0.0), axis=1, keepdims=True)

    @pl.when((e == 0) & (fc == 0))
    def _():
        out_ref[...] = jnp.zeros_like(out_ref)

    @pl.when(fc == 0)
    def _():
        out_ref[...] += g * b2_ref[0]

    out_ref[...] += g * eo


def _scatter_body(gidx_ref, os_ref, out_ref):
    i = pl.program_id(0)
    row = i * _TS5 + jax.lax.broadcasted_iota(jnp.int32, (_TS5, _NSEL), 0)
    onehot = (row == gidx_ref[...]).astype(jnp.float32)
    out_ref[...] = jnp.dot(onehot, os_ref[...],
                           preferred_element_type=jnp.float32)


@jax.jit
def kernel(x, gate_w, gate_b, local_w, local_b, W1, b1, W2, b2):
    xf = x.reshape(_BS, _D)
    gb2 = gate_b.reshape(1, _E)
    lb2 = local_b.reshape(1, 1)

    gs_flat, gidx, x_sel, gs_sel = pl.pallas_call(
        _gating_body,
        in_specs=[
            pl.BlockSpec((_BS, _D), lambda: (0, 0)),
            pl.BlockSpec((_D, _E), lambda: (0, 0)),
            pl.BlockSpec((1, _E), lambda: (0, 0)),
            pl.BlockSpec((_D, 1), lambda: (0, 0)),
            pl.BlockSpec((1, 1), lambda: (0, 0)),
        ],
        out_specs=[
            pl.BlockSpec((_BS, _E), lambda: (0, 0)),
            pl.BlockSpec((1, _NSEL), lambda: (0, 0)),
            pl.BlockSpec((_NSEL, _D), lambda: (0, 0)),
            pl.BlockSpec((_NSEL, _E), lambda: (0, 0)),
        ],
        out_shape=[
            jax.ShapeDtypeStruct((_BS, _E), jnp.float32),
            jax.ShapeDtypeStruct((1, _NSEL), jnp.int32),
            jax.ShapeDtypeStruct((_NSEL, _D), jnp.float32),
            jax.ShapeDtypeStruct((_NSEL, _E), jnp.float32),
        ],
    )(xf, gate_w, gb2, local_w, lb2)

    out_sel = pl.pallas_call(
        _expert_body,
        grid=(_E, _F // _FC),
        in_specs=[
            pl.BlockSpec((_NSEL, _D), lambda e, f: (0, 0)),
            pl.BlockSpec((_NSEL, _E), lambda e, f: (0, 0)),
            pl.BlockSpec((1, _D, _FC), lambda e, f: (e, 0, f)),
            pl.BlockSpec((1, 1, _FC), lambda e, f: (e, 0, f)),
            pl.BlockSpec((1, _FC, _F), lambda e, f: (e, f, 0)),
            pl.BlockSpec((1, 1, _F), lambda e, f: (e, 0, 0)),
        ],
        out_specs=pl.BlockSpec((_NSEL, _F), lambda e, f: (0, 0)),
        out_shape=jax.ShapeDtypeStruct((_NSEL, _F), jnp.float32),
    )(x_sel, gs_sel, W1, b1.reshape(_E, 1, _F), W2, b2.reshape(_E, 1, _F))

    out_flat = pl.pallas_call(
        _scatter_body,
        grid=(_BS // _TS5,),
        in_specs=[
            pl.BlockSpec((1, _NSEL), lambda i: (0, 0)),
            pl.BlockSpec((_NSEL, _F), lambda i: (0, 0)),
        ],
        out_specs=pl.BlockSpec((_TS5, _F), lambda i: (i, 0)),
        out_shape=jax.ShapeDtypeStruct((_BS, _F), jnp.float32),
    )(gidx, out_sel)

    return out_flat.reshape(_B, _S, _F), gs_flat.reshape(_B, _S, _E)


# speedup vs baseline: 3.7209x; 1.0445x over previous
import jax
import jax.numpy as jnp
from jax.experimental import pallas as pl
from jax.experimental.pallas import tpu as pltpu

_B, _S, _D = 2, 2048, 1024
_E = 8
_F = 1024
_ACTIVE_K = 8
_BS = _B * _S
_NSEL = _B * _ACTIVE_K

_TSG = 1024
_NGG = _BS // _TSG
_FC = 512
_NFC = _F // _FC
_NES = _E * _NFC
_TSO = 512
_NSC = _BS // _TSO
_GT = _NGG + _NES + _NSC


def _top2_gs(logits):
    m = jnp.max(logits, axis=-1, keepdims=True)
    p = jnp.exp(logits - m)
    p = p / jnp.sum(p, axis=-1, keepdims=True)
    eidx = jax.lax.broadcasted_iota(jnp.int32, p.shape, 1)
    m1 = jnp.max(p, axis=-1, keepdims=True)
    i1 = jnp.min(jnp.where(p == m1, eidx, _E), axis=-1, keepdims=True)
    p2 = jnp.where(eidx == i1, -jnp.inf, p)
    m2 = jnp.max(p2, axis=-1, keepdims=True)
    i2 = jnp.min(jnp.where(p2 == m2, eidx, _E), axis=-1, keepdims=True)
    mask = (eidx == i1) | (eidx == i2)
    gs = jnp.where(mask, p, 0.0)
    return gs / (jnp.sum(gs, axis=-1, keepdims=True) + 1e-9)


def _body(x_ref, gw_ref, gb_ref, lw_ref, lb_ref,
          w1_ref, b1_ref, w2_ref, b2_ref,
          gs_ref, out_ref,
          xbig, ls_s, gidx_s, xsel_s, gsel_s, osel_s):
    g = pl.program_id(0)

    @pl.when(g < _NGG)
    def _gating():
        xt = x_ref[...]
        xbig[pl.ds(g * _TSG, _TSG), :] = xt
        logits = jnp.dot(xt, gw_ref[...], preferred_element_type=jnp.float32)
        gs_ref[...] = _top2_gs(logits + gb_ref[...])
        ls = jnp.dot(xt, lw_ref[...], preferred_element_type=jnp.float32)
        ls_s[pl.ds(g, 1), :] = jnp.transpose(ls + lb_ref[...])

    @pl.when(g == _NGG)
    def _route():
        pos = jax.lax.broadcasted_iota(jnp.int32, (1, _NSEL), 1)
        gidx = jnp.zeros((1, _NSEL), jnp.int32)
        rows_per_b = _S // _TSG
        for b in range(_B):
            work = ls_s[pl.ds(b * rows_per_b, rows_per_b), :]
            gcol = (b * _S
                    + jax.lax.broadcasted_iota(jnp.int32, work.shape, 0) * _TSG
                    + jax.lax.broadcasted_iota(jnp.int32, work.shape, 1))
            for k in range(_ACTIVE_K):
                lm = jnp.max(jnp.max(work, axis=0, keepdims=True),
                             axis=1, keepdims=True)
                gi = jnp.min(jnp.min(jnp.where(work == lm, gcol, _BS),
                                     axis=0, keepdims=True),
                             axis=1, keepdims=True)
                gidx = jnp.where(pos == (b * _ACTIVE_K + k), gi, gidx)
                work = jnp.where(gcol == gi, -jnp.inf, work)
        gidx_s[...] = gidx

        colr = jax.lax.broadcasted_iota(jnp.int32, (_NSEL, _BS), 1)
        onehot = (colr == jnp.transpose(gidx)).astype(jnp.float32)
        xsel = jnp.dot(onehot, xbig[...], preferred_element_type=jnp.float32)
        xsel_s[...] = xsel
        lg = jnp.dot(xsel, gw_ref[...], preferred_element_type=jnp.float32)
        gsel_s[...] = _top2_gs(lg + gb_ref[...])

    @pl.when((g >= _NGG) & (g < _NGG + _NES))
    def _expert():
        eg = g - _NGG
        e = eg // _NFC
        fc = eg % _NFC
        h = jnp.dot(xsel_s[...], w1_ref[0], preferred_element_type=jnp.float32)
        h = jnp.maximum(h + b1_ref[0], 0.0)
        eo = jnp.dot(h, w2_ref[0], preferred_element_type=jnp.float32)
        gss = gsel_s[...]
        eidx = jax.lax.broadcasted_iota(jnp.int32, gss.shape, 1)
        gcol = jnp.sum(jnp.where(eidx == e, gss, 0.0), axis=1, keepdims=True)

        @pl.when(eg == 0)
        def _():
            osel_s[...] = jnp.zeros_like(osel_s)

        @pl.when(fc == 0)
        def _():
            osel_s[...] += gcol * b2_ref[0]

        osel_s[...] += gcol * eo

    @pl.when(g >= _NGG + _NES)
    def _scatter():
        t = g - (_NGG + _NES)
        row = (t * _TSO
               + jax.lax.broadcasted_iota(jnp.int32, (_TSO, _NSEL), 0))
        onehot = (row == gidx_s[...]).astype(jnp.float32)
        out_ref[...] = jnp.dot(onehot, osel_s[...],
                               preferred_element_type=jnp.float32)


def _expert_idx(g):
    eg = jnp.clip(g - _NGG, 0, _NES - 1)
    return eg // _NFC, eg % _NFC


@jax.jit
def kernel(x, gate_w, gate_b, local_w, local_b, W1, b1, W2, b2):
    xf = x.reshape(_BS, _D)
    gb2 = gate_b.reshape(1, _E)
    lb2 = local_b.reshape(1, 1)

    gs_flat, out_flat = pl.pallas_call(
        _body,
        grid=(_GT,),
        in_specs=[
            pl.BlockSpec((_TSG, _D), lambda g: (jnp.minimum(g, _NGG - 1), 0)),
            pl.BlockSpec((_D, _E), lambda g: (0, 0)),
            pl.BlockSpec((1, _E), lambda g: (0, 0)),
            pl.BlockSpec((_D, 1), lambda g: (0, 0)),
            pl.BlockSpec((1, 1), lambda g: (0, 0)),
            pl.BlockSpec((1, _D, _FC),
                         lambda g: (_expert_idx(g)[0], 0, _expert_idx(g)[1])),
            pl.BlockSpec((1, 1, _FC),
                         lambda g: (_expert_idx(g)[0], 0, _expert_idx(g)[1])),
            pl.BlockSpec((1, _FC, _F),
                         lambda g: (_expert_idx(g)[0], _expert_idx(g)[1], 0)),
            pl.BlockSpec((1, 1, _F), lambda g: (_expert_idx(g)[0], 0, 0)),
        ],
        out_specs=[
            pl.BlockSpec((_TSG, _E), lambda g: (jnp.minimum(g, _NGG - 1), 0)),
            pl.BlockSpec((_TSO, _F),
                         lambda g: (jnp.clip(g - (_NGG + _NES), 0, _NSC - 1), 0)),
        ],
        out_shape=[
            jax.ShapeDtypeStruct((_BS, _E), jnp.float32),
            jax.ShapeDtypeStruct((_BS, _F), jnp.float32),
        ],
        scratch_shapes=[
            pltpu.VMEM((_BS, _D), jnp.float32),
            pltpu.VMEM((_NGG, _TSG), jnp.float32),
            pltpu.VMEM((1, _NSEL), jnp.int32),
            pltpu.VMEM((_NSEL, _D), jnp.float32),
            pltpu.VMEM((_NSEL, _E), jnp.float32),
            pltpu.VMEM((_NSEL, _F), jnp.float32),
        ],
    )(xf, gate_w, gb2, local_w, lb2,
      W1, b1.reshape(_E, 1, _F), W2, b2.reshape(_E, 1, _F))

    return out_flat.reshape(_B, _S, _F), gs_flat.reshape(_B, _S, _E)


# speedup vs baseline: 4.1237x; 1.1082x over previous
import jax
import jax.numpy as jnp
from jax.experimental import pallas as pl
from jax.experimental.pallas import tpu as pltpu

_B, _S, _D = 2, 2048, 1024
_E = 8
_F = 1024
_ACTIVE_K = 8
_BS = _B * _S
_NSEL = _B * _ACTIVE_K

_TSG = 1024
_NGG = _BS // _TSG
_HC = 512
_CPE = 4
_NES = _E * _CPE
_TSO = 512
_NSC = _BS // _TSO
_GT = _NGG + _NES + _NSC
_RING = 5


def _top2_gs(logits):
    m = jnp.max(logits, axis=-1, keepdims=True)
    p = jnp.exp(logits - m)
    p = p / jnp.sum(p, axis=-1, keepdims=True)
    eidx = jax.lax.broadcasted_iota(jnp.int32, p.shape, 1)
    m1 = jnp.max(p, axis=-1, keepdims=True)
    i1 = jnp.min(jnp.where(p == m1, eidx, _E), axis=-1, keepdims=True)
    p2 = jnp.where(eidx == i1, -jnp.inf, p)
    m2 = jnp.max(p2, axis=-1, keepdims=True)
    i2 = jnp.min(jnp.where(p2 == m2, eidx, _E), axis=-1, keepdims=True)
    mask = (eidx == i1) | (eidx == i2)
    gs = jnp.where(mask, p, 0.0)
    return gs / (m1 + m2 + 1e-9)


def _issue(i, op, w1_hbm, w2_hbm, wring, sem):
    e = i // _CPE
    c = jax.lax.rem(i, _CPE)
    slot = jax.lax.rem(i, _RING)

    @pl.when(c < 2)
    def _():
        cp = pltpu.make_async_copy(
            w1_hbm.at[e, pl.ds(c * _HC, _HC), :], wring.at[slot], sem.at[slot])
        cp.start() if op == "start" else cp.wait()

    @pl.when(c >= 2)
    def _():
        cp = pltpu.make_async_copy(
            w2_hbm.at[e, pl.ds((c - 2) * _HC, _HC), :], wring.at[slot],
            sem.at[slot])
        cp.start() if op == "start" else cp.wait()


def _body(x_ref, gw_ref, gb_ref, lw_ref, lb_ref,
          w1_hbm, b1_ref, w2_hbm, b2_ref,
          gs_ref, out_ref,
          xbig, ls_s, gidx_s, xsel_s, gsel_s, osel_s, hpre_s, hrelu_s,
          wring, sem):
    g = pl.program_id(0)

    @pl.when(g < _NES)
    def _prefetch():
        _issue(g, "start", w1_hbm, w2_hbm, wring, sem)

    @pl.when(g < _NGG)
    def _gating():
        xt = x_ref[...]
        xbig[pl.ds(g * _TSG, _TSG), :] = xt
        logits = jnp.dot(xt, gw_ref[...], preferred_element_type=jnp.float32)
        gs_ref[...] = _top2_gs(logits + gb_ref[...])
        ls = jnp.dot(xt, lw_ref[...], preferred_element_type=jnp.float32)
        ls_s[pl.ds(g, 1), :] = jnp.transpose(ls + lb_ref[...])

    @pl.when(g == _NGG)
    def _route():
        pos = jax.lax.broadcasted_iota(jnp.int32, (1, _NSEL), 1)
        gidx = jnp.zeros((1, _NSEL), jnp.int32)
        rows_per_b = _S // _TSG
        for b in range(_B):
            work = ls_s[pl.ds(b * rows_per_b, rows_per_b), :]
            gcol = (b * _S
                    + jax.lax.broadcasted_iota(jnp.int32, work.shape, 0) * _TSG
                    + jax.lax.broadcasted_iota(jnp.int32, work.shape, 1))
            for k in range(_ACTIVE_K):
                lm = jnp.max(jnp.max(work, axis=0, keepdims=True),
                             axis=1, keepdims=True)
                gi = jnp.min(jnp.min(jnp.where(work == lm, gcol, _BS),
                                     axis=0, keepdims=True),
                             axis=1, keepdims=True)
                gidx = jnp.where(pos == (b * _ACTIVE_K + k), gi, gidx)
                work = jnp.where(gcol == gi, -jnp.inf, work)
        gidx_s[...] = gidx

        colr = jax.lax.broadcasted_iota(jnp.int32, (_NSEL, _BS), 1)
        onehot = (colr == jnp.transpose(gidx)).astype(jnp.float32)
        xsel = jnp.dot(onehot, xbig[...], preferred_element_type=jnp.float32)
        xsel_s[...] = xsel
        lg = jnp.dot(xsel, gw_ref[...], preferred_element_type=jnp.float32)
        gsel_s[...] = _top2_gs(lg + gb_ref[...])

    @pl.when((g >= _NGG) & (g < _NGG + _NES))
    def _expert():
        j = g - _NGG
        e = j // _CPE
        c = jax.lax.rem(j, _CPE)
        slot = jax.lax.rem(j, _RING)
        _issue(j, "wait", w1_hbm, w2_hbm, wring, sem)
        wv = wring[pl.ds(slot, 1)][0]

        @pl.when(c == 0)
        def _():
            hpre_s[...] = jnp.dot(xsel_s[:, 0:_HC], wv,
                                  preferred_element_type=jnp.float32)

        @pl.when(c == 1)
        def _():
            hpre_s[...] += jnp.dot(xsel_s[:, _HC:_D], wv,
                                   preferred_element_type=jnp.float32)

        gss = gsel_s[...]
        eidx = jax.lax.broadcasted_iota(jnp.int32, gss.shape, 1)
        gcol = jnp.sum(jnp.where(eidx == e, gss, 0.0), axis=1, keepdims=True)

        @pl.when(c == 2)
        def _():
            b1v = b1_ref[pl.ds(e, 1)][0]
            hr = jnp.maximum(hpre_s[...] + b1v, 0.0)
            hrelu_s[...] = hr

            @pl.when(j == 2)
            def _():
                osel_s[...] = jnp.zeros_like(osel_s)

            b2v = b2_ref[pl.ds(e, 1)][0]
            osel_s[...] += gcol * (
                jnp.dot(hr[:, 0:_HC], wv, preferred_element_type=jnp.float32)
                + b2v)

        @pl.when(c == 3)
        def _():
            osel_s[...] += gcol * jnp.dot(
                hrelu_s[:, _HC:_F], wv, preferred_element_type=jnp.float32)

    @pl.when(g >= _NGG + _NES)
    def _scatter():
        t = g - (_NGG + _NES)
        row = (t * _TSO
               + jax.lax.broadcasted_iota(jnp.int32, (_TSO, _NSEL), 0))
        onehot = (row == gidx_s[...]).astype(jnp.float32)
        out_ref[...] = jnp.dot(onehot, osel_s[...],
                               preferred_element_type=jnp.float32)


@jax.jit
def kernel(x, gate_w, gate_b, local_w, local_b, W1, b1, W2, b2):
    xf = x.reshape(_BS, _D)
    gb2 = gate_b.reshape(1, _E)
    lb2 = local_b.reshape(1, 1)

    gs_flat, out_flat = pl.pallas_call(
        _body,
        grid=(_GT,),
        in_specs=[
            pl.BlockSpec((_TSG, _D), lambda g: (jnp.minimum(g, _NGG - 1), 0)),
            pl.BlockSpec((_D, _E), lambda g: (0, 0)),
            pl.BlockSpec((1, _E), lambda g: (0, 0)),
            pl.BlockSpec((_D, 1), lambda g: (0, 0)),
            pl.BlockSpec((1, 1), lambda g: (0, 0)),
            pl.BlockSpec(memory_space=pl.ANY),
            pl.BlockSpec((_E, 1, _F), lambda g: (0, 0, 0)),
            pl.BlockSpec(memory_space=pl.ANY),
            pl.BlockSpec((_E, 1, _F), lambda g: (0, 0, 0)),
        ],
        out_specs=[
            pl.BlockSpec((_TSG, _E), lambda g: (jnp.minimum(g, _NGG - 1), 0)),
            pl.BlockSpec((_TSO, _F),
                         lambda g: (jnp.clip(g - (_NGG + _NES), 0, _NSC - 1), 0)),
        ],
        out_shape=[
            jax.ShapeDtypeStruct((_BS, _E), jnp.float32),
            jax.ShapeDtypeStruct((_BS, _F), jnp.float32),
        ],
        scratch_shapes=[
            pltpu.VMEM((_BS, _D), jnp.float32),
            pltpu.VMEM((_NGG, _TSG), jnp.float32),
            pltpu.VMEM((1, _NSEL), jnp.int32),
            pltpu.VMEM((_NSEL, _D), jnp.float32),
            pltpu.VMEM((_NSEL, _E), jnp.float32),
            pltpu.VMEM((_NSEL, _F), jnp.float32),
            pltpu.VMEM((_NSEL, _F), jnp.float32),
            pltpu.VMEM((_NSEL, _F), jnp.float32),
            pltpu.VMEM((_RING, _HC, _F), jnp.float32),
            pltpu.SemaphoreType.DMA((_RING,)),
        ],
    )(xf, gate_w, gb2, local_w, lb2,
      W1, b1.reshape(_E, 1, _F), W2, b2.reshape(_E, 1, _F))

    return out_flat.reshape(_B, _S, _F), gs_flat.reshape(_B, _S, _E)


# speedup vs baseline: 4.2204x; 1.0234x over previous
import jax
import jax.numpy as jnp
from jax.experimental import pallas as pl
from jax.experimental.pallas import tpu as pltpu

_B, _S, _D = 2, 2048, 1024
_E = 8
_F = 1024
_ACTIVE_K = 8
_BS = _B * _S
_NSEL = _B * _ACTIVE_K

_TSG = 1024
_NGG = _BS // _TSG
_HC = 512
_CPE = 4
_NES = _E * _CPE
_TSO = 512
_NSC = _BS // _TSO
_GT = _NGG + _NES + _NSC
_RING = 8
_LEAD = 4


def _top2_gs(logits):
    m = jnp.max(logits, axis=-1, keepdims=True)
    p = jnp.exp(logits - m)
    p = p / jnp.sum(p, axis=-1, keepdims=True)
    eidx = jax.lax.broadcasted_iota(jnp.int32, p.shape, 1)
    m1 = jnp.max(p, axis=-1, keepdims=True)
    i1 = jnp.min(jnp.where(p == m1, eidx, _E), axis=-1, keepdims=True)
    p2 = jnp.where(eidx == i1, -jnp.inf, p)
    m2 = jnp.max(p2, axis=-1, keepdims=True)
    i2 = jnp.min(jnp.where(p2 == m2, eidx, _E), axis=-1, keepdims=True)
    mask = (eidx == i1) | (eidx == i2)
    gs = jnp.where(mask, p, 0.0)
    return gs / (m1 + m2 + 1e-9)


def _issue(i, op, w1_hbm, w2_hbm, wring, sem):
    e = i // _CPE
    c = jax.lax.rem(i, _CPE)
    slot = jax.lax.rem(i, _RING)

    @pl.when(c < 2)
    def _():
        cp = pltpu.make_async_copy(
            w1_hbm.at[e, pl.ds(c * _HC, _HC), :], wring.at[slot], sem.at[slot])
        cp.start() if op == "start" else cp.wait()

    @pl.when(c >= 2)
    def _():
        cp = pltpu.make_async_copy(
            w2_hbm.at[e, pl.ds((c - 2) * _HC, _HC), :], wring.at[slot],
            sem.at[slot])
        cp.start() if op == "start" else cp.wait()


def _body(x_ref, gw_ref, gb_ref, lw_ref, lb_ref,
          w1_hbm, b1_ref, w2_hbm, b2_ref,
          gs_ref, out_ref,
          xbig, ls_s, gidx_s, xsel_s, gsel_s, osel_s, hpre_s, hrelu_s,
          wring, sem):
    g = pl.program_id(0)

    @pl.when(g == 0)
    def _prime():
        for i in range(_LEAD):
            _issue(i, "start", w1_hbm, w2_hbm, wring, sem)

    @pl.when((g >= 1) & (g + _LEAD - 1 < _NES))
    def _prefetch():
        _issue(g + _LEAD - 1, "start", w1_hbm, w2_hbm, wring, sem)

    @pl.when(g < _NGG)
    def _gating():
        xt = x_ref[...]
        xbig[pl.ds(g * _TSG, _TSG), :] = xt
        logits = jnp.dot(xt, gw_ref[...], preferred_element_type=jnp.float32)
        gs_ref[...] = _top2_gs(logits + gb_ref[...])
        ls = jnp.dot(xt, lw_ref[...], preferred_element_type=jnp.float32)
        ls_s[pl.ds(g, 1), :] = jnp.transpose(ls + lb_ref[...])

    @pl.when(g == _NGG)
    def _route():
        pos = jax.lax.broadcasted_iota(jnp.int32, (1, _NSEL), 1)
        gidx = jnp.zeros((1, _NSEL), jnp.int32)
        rows_per_b = _S // _TSG
        for b in range(_B):
            work = ls_s[pl.ds(b * rows_per_b, rows_per_b), :]
            gcol = (b * _S
                    + jax.lax.broadcasted_iota(jnp.int32, work.shape, 0) * _TSG
                    + jax.lax.broadcasted_iota(jnp.int32, work.shape, 1))
            for k in range(_ACTIVE_K):
                lm = jnp.max(jnp.max(work, axis=0, keepdims=True),
                             axis=1, keepdims=True)
                gi = jnp.min(jnp.min(jnp.where(work == lm, gcol, _BS),
                                     axis=0, keepdims=True),
                             axis=1, keepdims=True)
                gidx = jnp.where(pos == (b * _ACTIVE_K + k), gi, gidx)
                work = jnp.where(gcol == gi, -jnp.inf, work)
        gidx_s[...] = gidx

        colr = jax.lax.broadcasted_iota(jnp.int32, (_NSEL, _BS), 1)
        onehot = (colr == jnp.transpose(gidx)).astype(jnp.float32)
        xsel = jnp.dot(onehot, xbig[...], preferred_element_type=jnp.float32)
        xsel_s[...] = xsel
        lg = jnp.dot(xsel, gw_ref[...], preferred_element_type=jnp.float32)
        gsel_s[...] = _top2_gs(lg + gb_ref[...])

    @pl.when((g >= _NGG) & (g < _NGG + _NES))
    def _expert():
        j = g - _NGG
        e = j // _CPE
        c = jax.lax.rem(j, _CPE)
        slot = jax.lax.rem(j, _RING)
        _issue(j, "wait", w1_hbm, w2_hbm, wring, sem)
        wv = wring[pl.ds(slot, 1)][0]

        @pl.when(c == 0)
        def _():
            hpre_s[...] = jnp.dot(xsel_s[:, 0:_HC], wv,
                                  preferred_element_type=jnp.float32)

        @pl.when(c == 1)
        def _():
            hpre_s[...] += jnp.dot(xsel_s[:, _HC:_D], wv,
                                   preferred_element_type=jnp.float32)

        gss = gsel_s[...]
        eidx = jax.lax.broadcasted_iota(jnp.int32, gss.shape, 1)
        gcol = jnp.sum(jnp.where(eidx == e, gss, 0.0), axis=1, keepdims=True)

        @pl.when(c == 2)
        def _():
            b1v = b1_ref[pl.ds(e, 1)][0]
            hr = jnp.maximum(hpre_s[...] + b1v, 0.0)
            hrelu_s[...] = hr

            @pl.when(j == 2)
            def _():
                osel_s[...] = jnp.zeros_like(osel_s)

            b2v = b2_ref[pl.ds(e, 1)][0]
            osel_s[...] += gcol * (
                jnp.dot(hr[:, 0:_HC], wv, preferred_element_type=jnp.float32)
                + b2v)

        @pl.when(c == 3)
        def _():
            osel_s[...] += gcol * jnp.dot(
                hrelu_s[:, _HC:_F], wv, preferred_element_type=jnp.float32)

    @pl.when(g >= _NGG + _NES)
    def _scatter():
        t = g - (_NGG + _NES)
        row = (t * _TSO
               + jax.lax.broadcasted_iota(jnp.int32, (_TSO, _NSEL), 0))
        onehot = (row == gidx_s[...]).astype(jnp.float32)
        out_ref[...] = jnp.dot(onehot, osel_s[...],
                               preferred_element_type=jnp.float32)


@jax.jit
def kernel(x, gate_w, gate_b, local_w, local_b, W1, b1, W2, b2):
    xf = x.reshape(_BS, _D)
    gb2 = gate_b.reshape(1, _E)
    lb2 = local_b.reshape(1, 1)

    gs_flat, out_flat = pl.pallas_call(
        _body,
        grid=(_GT,),
        in_specs=[
            pl.BlockSpec((_TSG, _D), lambda g: (jnp.minimum(g, _NGG - 1), 0)),
            pl.BlockSpec((_D, _E), lambda g: (0, 0)),
            pl.BlockSpec((1, _E), lambda g: (0, 0)),
            pl.BlockSpec((_D, 1), lambda g: (0, 0)),
            pl.BlockSpec((1, 1), lambda g: (0, 0)),
            pl.BlockSpec(memory_space=pl.ANY),
            pl.BlockSpec((_E, 1, _F), lambda g: (0, 0, 0)),
            pl.BlockSpec(memory_space=pl.ANY),
            pl.BlockSpec((_E, 1, _F), lambda g: (0, 0, 0)),
        ],
        out_specs=[
            pl.BlockSpec((_TSG, _E), lambda g: (jnp.minimum(g, _NGG - 1), 0)),
            pl.BlockSpec((_TSO, _F),
                         lambda g: (jnp.clip(g - (_NGG + _NES), 0, _NSC - 1), 0)),
        ],
        out_shape=[
            jax.ShapeDtypeStruct((_BS, _E), jnp.float32),
            jax.ShapeDtypeStruct((_BS, _F), jnp.float32),
        ],
        scratch_shapes=[
            pltpu.VMEM((_BS, _D), jnp.float32),
            pltpu.VMEM((_NGG, _TSG), jnp.float32),
            pltpu.VMEM((1, _NSEL), jnp.int32),
            pltpu.VMEM((_NSEL, _D), jnp.float32),
            pltpu.VMEM((_NSEL, _E), jnp.float32),
            pltpu.VMEM((_NSEL, _F), jnp.float32),
            pltpu.VMEM((_NSEL, _F), jnp.float32),
            pltpu.VMEM((_NSEL, _F), jnp.float32),
            pltpu.VMEM((_RING, _HC, _F), jnp.float32),
            pltpu.SemaphoreType.DMA((_RING,)),
        ],
    )(xf, gate_w, gb2, local_w, lb2,
      W1, b1.reshape(_E, 1, _F), W2, b2.reshape(_E, 1, _F))

    return out_flat.reshape(_B, _S, _F), gs_flat.reshape(_B, _S, _E)
